# baseline (speedup 1.0000x reference)
import functools

import jax
import jax.numpy as jnp
from jax import lax
from jax.experimental import pallas as pl
from jax.experimental.pallas import tpu as pltpu
from jax.experimental.pallas import tpu_sc as plsc

_NC = 2
_NS = 16
_NW = _NC * _NS


def _rsqrt_newton(x):
    i = lax.bitcast_convert_type(x, jnp.int32)
    y = lax.bitcast_convert_type(
        jnp.int32(0x5F3759DF) - lax.shift_right_arithmetic(i, 1), jnp.float32)
    for _ in range(3):
        y = y * (1.5 - 0.5 * x * y * y)
    return y


def _sc_gather_ln(table, ids, gamma, beta, d_model):
    b = ids.shape[0]
    b_per_w = b // _NW
    chunk = 32
    n_chunks = b_per_w // chunk
    n_lanes = d_model // 16
    mesh = plsc.VectorSubcoreMesh(
        core_axis_name="c", subcore_axis_name="s", num_cores=_NC,
        num_subcores=_NS)

    nbuf = 4

    @functools.partial(
        pl.kernel,
        out_type=jax.ShapeDtypeStruct((b, d_model), jnp.float32),
        mesh=mesh,
        scratch_types=[
            pltpu.VMEM((b_per_w,), jnp.int32),
            pltpu.VMEM((nbuf, chunk, d_model), jnp.float32),
            pltpu.VMEM((d_model,), jnp.float32),
            pltpu.VMEM((d_model,), jnp.float32),
            pltpu.SemaphoreType.DMA((nbuf,)),
            pltpu.SemaphoreType.DMA((nbuf,)),
        ],
    )
    def k(table_hbm, idx_hbm, g_hbm, bta_hbm, out_hbm, idx_v, rows_v, g_v,
          bta_v, gsem, wsem):
        wid = lax.axis_index("s") * _NC + lax.axis_index("c")
        base = wid * b_per_w
        pltpu.sync_copy(g_hbm, g_v)
        pltpu.sync_copy(bta_hbm, bta_v)
        pltpu.sync_copy(idx_hbm.at[pl.ds(base, b_per_w)], idx_v)

        def start(c, slot):
            return pltpu.async_copy(
                table_hbm.at[idx_v.at[pl.ds(c * chunk, chunk)]],
                rows_v.at[slot], gsem.at[slot])

        def wait_gather(slot):
            pltpu.make_async_copy(table_hbm.at[idx_v.at[pl.ds(0, chunk)]],
                                  rows_v.at[slot], gsem.at[slot]).wait()

        half = chunk // 2

        def start_write_half(c, slot, h):
            pltpu.async_copy(
                rows_v.at[slot, pl.ds(h * half, half)],
                out_hbm.at[pl.ds(base + c * chunk + h * half, half)],
                wsem.at[slot])

        def wait_write(slot):
            pltpu.make_async_copy(rows_v.at[slot],
                                  out_hbm.at[pl.ds(base, chunk)],
                                  wsem.at[slot]).wait()

        def lane_sum(v):
            v = v + lax.rev(v, (0,))
            parts = [v[i] for i in range(8)]
            while len(parts) > 1:
                parts = [parts[2 * i] + parts[2 * i + 1]
                         for i in range(len(parts) // 2)]
            return parts[0]

        n_unroll = 8

        def ln_rows(r0, slot):
            stats = []
            for q in range(n_unroll):
                r = r0 * n_unroll + q
                acc_s = rows_v[slot, r, pl.ds(0, 16)]
                acc_q = acc_s * acc_s
                for j in range(1, n_lanes):
                    v = rows_v[slot, r, pl.ds(j * 16, 16)]
                    acc_s = acc_s + v
                    acc_q = acc_q + v * v
                mu = lane_sum(acc_s) * (1.0 / d_model)
                var = lane_sum(acc_q) * (1.0 / d_model) - mu * mu
                stats.append((mu, _rsqrt_newton(var + 1e-5)))
            for j in range(n_lanes):
                sl = pl.ds(j * 16, 16)
                g_blk = g_v[sl]
                b_blk = bta_v[sl]
                for q in range(n_unroll):
                    r = r0 * n_unroll + q
                    mu, rstd = stats[q]
                    v = rows_v[slot, r, sl]
                    y = (v - mu) * (rstd * g_blk) + b_blk
                    rows_v[slot, r, sl] = jnp.maximum(y, 0.0)
            return 0

        start(0, 0)
        start(1, 1)

        def phase(c, _):
            slot = c % nbuf
            wait_gather(slot)
            c2 = c + 2
            slot2 = c2 % nbuf

            @pl.when(c2 < n_chunks)
            def _():
                @pl.when(c >= 2)
                def _():
                    wait_write(slot2)
                start(c2, slot2)

            groups_per_half = half // n_unroll
            for h in range(2):
                lax.fori_loop(
                    h * groups_per_half, (h + 1) * groups_per_half,
                    lambda r0, __: ln_rows(r0, slot), 0)
                start_write_half(c, slot, h)
            return 0

        lax.fori_loop(0, n_chunks, phase, 0)
        for tail in range(n_chunks - nbuf, n_chunks):
            wait_write(tail % nbuf)

    return k(table, ids, gamma, beta)


def _ln_relu(v, gamma, beta):
    eps = 1e-5
    mu = jnp.mean(v, axis=-1, keepdims=True)
    var = jnp.mean(jnp.square(v - mu), axis=-1, keepdims=True)
    y = (v - mu) * lax.rsqrt(var + eps) * gamma + beta
    return jnp.maximum(y, 0.0)


def _tc_ratio_body(x_ref, w_ref, b_ref, rg_ref, rb_ref, ratio_ref):
    r = lax.dot_general(x_ref[...], w_ref[...], (((1,), (1,)), ((), ())),
                        preferred_element_type=jnp.float32)
    r = r + b_ref[...]
    ratio_ref[...] = _ln_relu(r, rg_ref[...], rb_ref[...])


def kernel(canvas_cat_ids, canvas_ratio_feat, cat_table, cat_ln_g, cat_ln_b,
           ratio_W, ratio_b, ratio_ln_g, ratio_ln_b):
    b, in_feat = canvas_ratio_feat.shape
    d_model = cat_table.shape[1]
    ids = canvas_cat_ids.astype(jnp.int32)

    cat = _sc_gather_ln(cat_table, ids, cat_ln_g, cat_ln_b, d_model)

    bn = 1024
    grid = (b // bn,)
    row_spec = pl.BlockSpec((bn, d_model), lambda i: (i, 0))
    vec_spec = pl.BlockSpec((1, d_model), lambda i: (0, 0))
    full_w = pl.BlockSpec((d_model, in_feat), lambda i: (0, 0))

    ratio = pl.pallas_call(
        _tc_ratio_body,
        grid=grid,
        in_specs=[
            pl.BlockSpec((bn, in_feat), lambda i: (i, 0)),
            full_w,
            vec_spec, vec_spec, vec_spec,
        ],
        out_specs=row_spec,
        out_shape=jax.ShapeDtypeStruct((b, d_model), jnp.float32),
    )(
        canvas_ratio_feat, ratio_W,
        ratio_b.reshape(1, -1), ratio_ln_g.reshape(1, -1),
        ratio_ln_b.reshape(1, -1),
    )
    return (cat, ratio)


# speedup vs baseline: 1.5172x; 1.5172x over previous
import functools

import jax
import jax.numpy as jnp
from jax import lax
from jax.experimental import pallas as pl
from jax.experimental.pallas import tpu as pltpu
from jax.experimental.pallas import tpu_sc as plsc

_NC = 2
_NS = 16
_NW = _NC * _NS


def _rsqrt_newton(x):
    i = lax.bitcast_convert_type(x, jnp.int32)
    y = lax.bitcast_convert_type(
        jnp.int32(0x5F3759DF) - lax.shift_right_arithmetic(i, 1), jnp.float32)
    for _ in range(3):
        y = y * (1.5 - 0.5 * x * y * y)
    return y


def _sc_gather_ln(table, ids, gamma, beta, d_model):
    b = ids.shape[0]
    b_per_w = b // _NW
    chunk = 16
    n_chunks = b_per_w // chunk
    n_lanes = d_model // 16
    mesh = plsc.VectorSubcoreMesh(
        core_axis_name="c", subcore_axis_name="s", num_cores=_NC,
        num_subcores=_NS)

    nbuf = 8
    lead = 4

    @functools.partial(
        pl.kernel,
        out_type=jax.ShapeDtypeStruct((b, d_model), jnp.float32),
        mesh=mesh,
        scratch_types=[
            pltpu.VMEM((b_per_w,), jnp.int32),
            pltpu.VMEM((nbuf, chunk, d_model), jnp.float32),
            pltpu.VMEM((d_model,), jnp.float32),
            pltpu.VMEM((d_model,), jnp.float32),
            pltpu.SemaphoreType.DMA((nbuf,)),
            pltpu.SemaphoreType.DMA((nbuf,)),
        ],
    )
    def k(table_hbm, idx_hbm, g_hbm, bta_hbm, out_hbm, idx_v, rows_v, g_v,
          bta_v, gsem, wsem):
        wid = lax.axis_index("s") * _NC + lax.axis_index("c")
        base = wid * b_per_w
        pltpu.sync_copy(g_hbm, g_v)
        pltpu.sync_copy(bta_hbm, bta_v)
        pltpu.sync_copy(idx_hbm.at[pl.ds(base, b_per_w)], idx_v)

        def start(c, slot):
            return pltpu.async_copy(
                table_hbm.at[idx_v.at[pl.ds(c * chunk, chunk)]],
                rows_v.at[slot], gsem.at[slot])

        def wait_gather(slot):
            pltpu.make_async_copy(table_hbm.at[idx_v.at[pl.ds(0, chunk)]],
                                  rows_v.at[slot], gsem.at[slot]).wait()

        def start_write(c, slot):
            pltpu.async_copy(rows_v.at[slot],
                             out_hbm.at[pl.ds(base + c * chunk, chunk)],
                             wsem.at[slot])

        def wait_write(slot):
            pltpu.make_async_copy(rows_v.at[slot],
                                  out_hbm.at[pl.ds(base, chunk)],
                                  wsem.at[slot]).wait()

        def lane_sum(v):
            v = v + lax.rev(v, (0,))
            parts = [v[i] for i in range(8)]
            while len(parts) > 1:
                parts = [parts[2 * i] + parts[2 * i + 1]
                         for i in range(len(parts) // 2)]
            return parts[0]

        n_unroll = 8

        def ln_rows(r0, slot):
            stats = []
            for q in range(n_unroll):
                r = r0 * n_unroll + q
                acc_s = rows_v[slot, r, pl.ds(0, 16)]
                acc_q = acc_s * acc_s
                for j in range(1, n_lanes):
                    v = rows_v[slot, r, pl.ds(j * 16, 16)]
                    acc_s = acc_s + v
                    acc_q = acc_q + v * v
                mu = lane_sum(acc_s) * (1.0 / d_model)
                var = lane_sum(acc_q) * (1.0 / d_model) - mu * mu
                stats.append((mu, _rsqrt_newton(var + 1e-5)))
            for j in range(n_lanes):
                sl = pl.ds(j * 16, 16)
                g_blk = g_v[sl]
                b_blk = bta_v[sl]
                for q in range(n_unroll):
                    r = r0 * n_unroll + q
                    mu, rstd = stats[q]
                    v = rows_v[slot, r, sl]
                    y = (v - mu) * (rstd * g_blk) + b_blk
                    rows_v[slot, r, sl] = jnp.maximum(y, 0.0)
            return 0

        for p in range(4):
            start(p, p)

        def phase(c, _):
            slot = c % nbuf
            wait_gather(slot)
            c2 = c + lead
            slot2 = c2 % nbuf

            @pl.when(c2 < n_chunks)
            def _():
                @pl.when(c >= nbuf - lead)
                def _():
                    wait_write(slot2)
                start(c2, slot2)

            lax.fori_loop(0, chunk // n_unroll,
                          lambda r0, __: ln_rows(r0, slot), 0)
            start_write(c, slot)
            return 0

        lax.fori_loop(0, n_chunks, phase, 0)
        for tail in range(n_chunks - nbuf, n_chunks):
            wait_write(tail % nbuf)

    return k(table, ids, gamma, beta)


def _ln_relu(v, gamma, beta):
    eps = 1e-5
    mu = jnp.mean(v, axis=-1, keepdims=True)
    var = jnp.mean(jnp.square(v - mu), axis=-1, keepdims=True)
    y = (v - mu) * lax.rsqrt(var + eps) * gamma + beta
    return jnp.maximum(y, 0.0)


def _tc_ratio_body(x_ref, w_ref, b_ref, rg_ref, rb_ref, ratio_ref):
    r = lax.dot_general(x_ref[...], w_ref[...], (((1,), (1,)), ((), ())),
                        preferred_element_type=jnp.float32)
    r = r + b_ref[...]
    ratio_ref[...] = _ln_relu(r, rg_ref[...], rb_ref[...])


def kernel(canvas_cat_ids, canvas_ratio_feat, cat_table, cat_ln_g, cat_ln_b,
           ratio_W, ratio_b, ratio_ln_g, ratio_ln_b):
    b, in_feat = canvas_ratio_feat.shape
    d_model = cat_table.shape[1]
    ids = canvas_cat_ids.astype(jnp.int32)

    cat = _sc_gather_ln(cat_table, ids, cat_ln_g, cat_ln_b, d_model)

    bn = 1024
    grid = (b // bn,)
    row_spec = pl.BlockSpec((bn, d_model), lambda i: (i, 0))
    vec_spec = pl.BlockSpec((1, d_model), lambda i: (0, 0))
    full_w = pl.BlockSpec((d_model, in_feat), lambda i: (0, 0))

    ratio = pl.pallas_call(
        _tc_ratio_body,
        grid=grid,
        in_specs=[
            pl.BlockSpec((bn, in_feat), lambda i: (i, 0)),
            full_w,
            vec_spec, vec_spec, vec_spec,
        ],
        out_specs=row_spec,
        out_shape=jax.ShapeDtypeStruct((b, d_model), jnp.float32),
    )(
        canvas_ratio_feat, ratio_W,
        ratio_b.reshape(1, -1), ratio_ln_g.reshape(1, -1),
        ratio_ln_b.reshape(1, -1),
    )
    return (cat, ratio)
